# Initial kernel scaffold; baseline (speedup 1.0000x reference)
import jax
import jax.numpy as jnp
from jax.experimental import pallas as pl


def kernel(metadata_a, metadata_b, x_a, edge_index_a, batch_vector_a, x_b, edge_index_b, batch_vector_b, W_fc1, b_fc1, W_fc2, b_fc2, W_g1, b_g1, W_g2, b_g2, W_comb, b_comb, W_out, b_out):
    raise NotImplementedError("write your pallas kernel here")


# speedup vs baseline: 3.1491x; 3.1491x over previous
import functools

import jax
import jax.numpy as jnp
from jax.experimental import pallas as pl
from jax.experimental.pallas import tpu as pltpu


def _mlp_body(md_ref, w1_ref, b1_ref, w2_ref, b2_ref, out_ref):
    h = jax.lax.dot(
        md_ref[...].astype(jnp.bfloat16),
        w1_ref[...].astype(jnp.bfloat16),
        preferred_element_type=jnp.float32,
    )
    h = jnp.maximum(h + b1_ref[...], 0.0)
    h2 = jax.lax.dot(
        h.astype(jnp.bfloat16),
        w2_ref[...].astype(jnp.bfloat16),
        preferred_element_type=jnp.float32,
    )
    out_ref[...] = jnp.maximum(h2 + b2_ref[...], 0.0)


def _metadata_mlp(md, W1, b1, W2, b2):
    B, M = md.shape
    BLK = 128
    grid = (B // BLK,)
    return pl.pallas_call(
        _mlp_body,
        grid=grid,
        in_specs=[
            pl.BlockSpec((BLK, M), lambda i: (i, 0)),
            pl.BlockSpec((M, 128), lambda i: (0, 0)),
            pl.BlockSpec((1, 128), lambda i: (0, 0)),
            pl.BlockSpec((128, 128), lambda i: (0, 0)),
            pl.BlockSpec((1, 128), lambda i: (0, 0)),
        ],
        out_specs=pl.BlockSpec((BLK, 128), lambda i: (i, 0)),
        out_shape=jax.ShapeDtypeStruct((B, 128), jnp.float32),
        compiler_params=pltpu.CompilerParams(
            dimension_semantics=("arbitrary",),
        ),
    )(md, W1, b1.reshape(1, 128), W2, b2.reshape(1, 128))


def _head_body(ma_ref, mb_ref, pa_ref, pb_ref, ca_ref, cb_ref,
               wc_ref, bc_ref, wo_ref, bo_ref, out_ref):
    pa = pa_ref[...] / jnp.maximum(ca_ref[...], 1.0)
    pb = pb_ref[...] / jnp.maximum(cb_ref[...], 1.0)
    comb = jnp.concatenate([ma_ref[...], mb_ref[...], pa, pb], axis=1)
    c = jnp.maximum(
        jax.lax.dot(comb, wc_ref[...], preferred_element_type=jnp.float32)
        + bc_ref[...], 0.0)
    o = jax.lax.dot(c, wo_ref[...], preferred_element_type=jnp.float32) + bo_ref[...]
    out_ref[...] = jax.nn.sigmoid(o)


def _head(ma, mb, pa_sum, pb_sum, cnt_a, cnt_b, Wc, bc, Wo, bo):
    B = ma.shape[0]
    return pl.pallas_call(
        _head_body,
        grid=(1,),
        in_specs=[pl.BlockSpec(x.shape, lambda i: (0,) * x.ndim) for x in
                  (ma, mb, pa_sum, pb_sum, cnt_a, cnt_b, Wc)]
        + [pl.BlockSpec((1, 128), lambda i: (0, 0)),
           pl.BlockSpec((128, 1), lambda i: (0, 0)),
           pl.BlockSpec((1, 1), lambda i: (0, 0))],
        out_specs=pl.BlockSpec((B, 1), lambda i: (0, 0)),
        out_shape=jax.ShapeDtypeStruct((B, 1), jnp.float32),
    )(ma, mb, pa_sum, pb_sum, cnt_a, cnt_b, Wc,
      bc.reshape(1, 128), Wo, bo.reshape(1, 1))


def _gcn_pool_side(x, edge_index, batch, W1, b1, W2, b2, B):
    N, F = x.shape
    src = edge_index[0].astype(jnp.int32)
    dst = edge_index[1].astype(jnp.int32)
    batch = batch.astype(jnp.int32)

    deg = jnp.zeros((N,), jnp.float32).at[dst].add(1.0) + 1.0
    dinv = jax.lax.rsqrt(deg)[:, None]

    g0 = x * dinv
    s1 = jnp.zeros((N, F), jnp.float32).at[dst].add(g0[src])
    out1 = dinv * s1 + dinv * dinv * x
    h1 = jnp.maximum(out1 @ W1 + b1, 0.0)

    g1 = h1 * dinv
    s2 = jnp.zeros((N, 128), jnp.float32).at[dst].add(g1[src])
    out2 = dinv * (s2 + g1)
    h2 = jnp.maximum(out2 @ W2 + b2, 0.0)

    psum = jax.ops.segment_sum(h2, batch, num_segments=B)
    cnt = jax.ops.segment_sum(jnp.ones((N,), jnp.float32), batch, num_segments=B)
    return psum, cnt[:, None]


def kernel(metadata_a, metadata_b, x_a, edge_index_a, batch_vector_a,
           x_b, edge_index_b, batch_vector_b, W_fc1, b_fc1, W_fc2, b_fc2,
           W_g1, b_g1, W_g2, b_g2, W_comb, b_comb, W_out, b_out):
    B = metadata_a.shape[0]
    ma = _metadata_mlp(metadata_a, W_fc1, b_fc1, W_fc2, b_fc2)
    mb = _metadata_mlp(metadata_b, W_fc1, b_fc1, W_fc2, b_fc2)
    pa_sum, cnt_a = _gcn_pool_side(x_a, edge_index_a, batch_vector_a,
                                   W_g1, b_g1, W_g2, b_g2, B)
    pb_sum, cnt_b = _gcn_pool_side(x_b, edge_index_b, batch_vector_b,
                                   W_g1, b_g1, W_g2, b_g2, B)
    return _head(ma, mb, pa_sum, pb_sum, cnt_a, cnt_b,
                 W_comb, b_comb, W_out, b_out)
